# baseline (speedup 1.0000x reference)
import numpy as np
import jax
import jax.numpy as jnp
from jax.experimental import pallas as pl
from jax.experimental.pallas import tpu as pltpu

_NJ = 14
_COL = 14
_SIGMA = 1.0

_ROWS_PER_BLK = 896
_NBLK = (8192 * _NJ) // _ROWS_PER_BLK


def _gauss_reflect_matrix_np(n, sigma):
    radius = int(4.0 * sigma + 0.5)
    x = np.arange(-radius, radius + 1)
    w = np.exp(-0.5 * (x / sigma) ** 2)
    w = w / w.sum()
    K = np.zeros((n, n), dtype=np.float64)
    for i in range(n):
        for d in range(-radius, radius + 1):
            j = i + d
            if j < 0:
                j = -j - 1
            elif j >= n:
                j = 2 * n - 1 - j
            K[i, j] += w[d + radius]
    return K


def _build_table_np():
    K = _gauss_reflect_matrix_np(_COL, _SIGMA).astype(np.float32)
    tbl = np.zeros((_COL * _COL, _COL * _COL + 1), dtype=np.float32)
    for yi in range(_COL):
        for xi in range(_COL):
            u = K[:, yi]
            w = K[:, xi]
            f = np.outer(u, w).astype(np.float32)
            mn = f.min()
            mx = f.max()
            denom = max(mx - mn, 1e-12)
            heat = ((f - mn) / denom).astype(np.float32)
            tbl[yi * _COL + xi, : _COL * _COL] = heat.reshape(-1)
            tbl[yi * _COL + xi, _COL * _COL] = 1.0
    return tbl


_TBL_NP = _build_table_np()


def _mse_heatmap_kernel(tx_ref, ty_ref, vv_ref, h_ref, tbl_ref,
                        tt_ref, psum_ref, cnt_ref):
    R = _ROWS_PER_BLK
    M = _COL * _COL

    tx = tx_ref[0]
    ty = ty_ref[0]
    vv = vv_ref[0]
    xi = jnp.trunc(tx * _COL).astype(jnp.int32)
    yi = jnp.trunc(ty * _COL).astype(jnp.int32)
    valid = ((xi >= 0) & (xi <= _COL - 1) & (yi >= 0) & (yi <= _COL - 1))
    vis = (vv == 1) & valid
    code = jnp.where(vis, yi * _COL + xi, -1)

    c_iota = jax.lax.broadcasted_iota(jnp.int32, (M, R), 0)
    ohct = (c_iota == code).astype(jnp.float32)

    tt_aug = jax.lax.dot_general(
        ohct, tbl_ref[...],
        dimension_numbers=(((0,), (0,)), ((), ())),
        preferred_element_type=jnp.float32)
    tt = tt_aug[:, :M]
    vis_col = tt_aug[:, M:M + 1]
    tt_ref[...] = tt

    diff = h_ref[...] - tt
    lane = jax.lax.broadcasted_iota(jnp.int32, (R, M), 1)
    row0_invis = (lane < _COL) & (vis_col < 0.5)
    diff = jnp.where(row0_invis, 0.0, diff)
    psum_ref[...] = jnp.sum(diff * diff, axis=0, keepdims=True)[None]
    cnt = jnp.sum(vis_col, axis=0, keepdims=True)
    cnt_ref[...] = jnp.broadcast_to(cnt, (1, M))[None]


def kernel(o, h, t, v):
    del o
    B = h.shape[0]
    rows = B * _NJ
    M = _COL * _COL

    hr = h.reshape(rows, M)
    tx = t[..., 0].reshape(_NBLK, 1, _ROWS_PER_BLK)
    ty = t[..., 1].reshape(_NBLK, 1, _ROWS_PER_BLK)
    vv = v[..., 0].reshape(_NBLK, 1, _ROWS_PER_BLK)
    tbl = jnp.asarray(_TBL_NP)

    grid = (_NBLK,)
    lane_spec = pl.BlockSpec((1, 1, _ROWS_PER_BLK), lambda i: (i, 0, 0))
    tt_r, psums, cnts = pl.pallas_call(
        _mse_heatmap_kernel,
        grid=grid,
        in_specs=[
            lane_spec, lane_spec, lane_spec,
            pl.BlockSpec((_ROWS_PER_BLK, M), lambda i: (i, 0)),
            pl.BlockSpec((M, M + 1), lambda i: (0, 0)),
        ],
        out_specs=[
            pl.BlockSpec((_ROWS_PER_BLK, M), lambda i: (i, 0)),
            pl.BlockSpec((1, 1, M), lambda i: (i, 0, 0)),
            pl.BlockSpec((1, 1, M), lambda i: (i, 0, 0)),
        ],
        out_shape=[
            jax.ShapeDtypeStruct((rows, M), jnp.float32),
            jax.ShapeDtypeStruct((_NBLK, 1, M), jnp.float32),
            jax.ShapeDtypeStruct((_NBLK, 1, M), jnp.float32),
        ],
        compiler_params=pltpu.CompilerParams(
            dimension_semantics=("parallel",),
        ),
    )(tx, ty, vv, hr, tbl)

    tt = tt_r.reshape(B, _NJ, _COL, _COL)
    cnt_total = jnp.sum(cnts[:, 0, 0])
    d1 = jnp.sum(psums) / cnt_total
    return d1, tt


# speedup vs baseline: 24.3439x; 24.3439x over previous
import numpy as np
import jax
import jax.numpy as jnp
from jax.experimental import pallas as pl
from jax.experimental.pallas import tpu as pltpu

_NJ = 14
_COL = 14
_SIGMA = 1.0

_BB = 4096
_NB = 8192 // _BB
_JPB = 2
_NJB = _NJ // _JPB


def _gauss_reflect_matrix_np(n, sigma):
    radius = int(4.0 * sigma + 0.5)
    x = np.arange(-radius, radius + 1)
    w = np.exp(-0.5 * (x / sigma) ** 2)
    w = w / w.sum()
    K = np.zeros((n, n), dtype=np.float64)
    for i in range(n):
        for d in range(-radius, radius + 1):
            j = i + d
            if j < 0:
                j = -j - 1
            elif j >= n:
                j = 2 * n - 1 - j
            K[i, j] += w[d + radius]
    return K


def _build_kbig_np():
    K = _gauss_reflect_matrix_np(_COL, _SIGMA).astype(np.float32)
    kbig = np.zeros((32, 28), dtype=np.float32)
    kbig[0:14, 0:14] = K
    kbig[14, 0:14] = K.min(axis=0)
    kbig[15, 0:14] = K.max(axis=0)
    kbig[16:30, 14:28] = K
    kbig[30, 14:28] = K.min(axis=0)
    kbig[31, 14:28] = K.max(axis=0)
    return kbig


_KBIG_NP = _build_kbig_np()


def _fused_kernel(tT_ref, vT_ref, h_ref, kbig_ref, tt_ref, psum_ref, cnt_ref):
    jj = pl.program_id(1)
    parts = []
    cnts = []
    for jo in range(_JPB):
        _one_joint(jj * _JPB + jo, jo, tT_ref, vT_ref, h_ref, kbig_ref,
                   tt_ref, parts, cnts)

    @pl.when(jj == 0)
    def _init():
        psum_ref[...] = jnp.zeros_like(psum_ref)
        cnt_ref[...] = jnp.zeros_like(cnt_ref)

    psum_ref[...] += sum(parts)
    cnt_ref[...] += sum(cnts).reshape(1, 1, _BB)


def _one_joint(j, jo, tT_ref, vT_ref, h_ref, kbig_ref, tt_ref, parts, cnts):
    tx = tT_ref[j, 0:1, :]
    ty = tT_ref[j, 1:2, :]
    vv = vT_ref[j, 0:1, :]

    xi = jnp.trunc(tx * _COL).astype(jnp.int32)
    yi = jnp.trunc(ty * _COL).astype(jnp.int32)
    valid = (xi >= 0) & (xi <= _COL - 1) & (yi >= 0) & (yi <= _COL - 1)
    vis = (vv == 1) & valid

    i_iota = jax.lax.broadcasted_iota(jnp.int32, (28, _BB), 0)
    tgt = jnp.where(i_iota < _COL, yi, xi + _COL)
    oh = jnp.where(i_iota == tgt, 1.0, 0.0)

    s = jax.lax.dot_general(
        kbig_ref[...], oh,
        dimension_numbers=(((1,), (0,)), ((), ())),
        preferred_element_type=jnp.float32)
    u = s[0:14, :]
    mnu = s[14:15, :]
    mxu = s[15:16, :]
    w = s[16:30, :]
    mnw = s[30:31, :]
    mxw = s[31:32, :]

    mn = mnu * mnw
    denom = jnp.maximum(mxu * mxw - mn, 1e-12)
    visf = jnp.where(vis, 1.0, 0.0)
    scale = visf / denom
    us = u * scale
    c3 = (mn * scale).reshape(1, 1, _BB)

    us3 = us.reshape(14, 1, _BB)
    w3 = w.reshape(1, 14, _BB)
    tt = us3 * w3 - c3
    tt_ref[jo * _COL:(jo + 1) * _COL] = tt

    diff = h_ref[jo * _COL:(jo + 1) * _COL] - tt
    sq = diff * diff
    invis3 = (visf == 0.0).reshape(1, 1, _BB)
    sq0 = jnp.where(invis3, 0.0, sq[0:1])
    part = jnp.sum(sq[1:], axis=(0, 1), keepdims=True) + \
        jnp.sum(sq0, axis=(0, 1), keepdims=True)
    parts.append(part)
    cnts.append(visf)


def kernel(o, h, t, v):
    del o
    B = h.shape[0]

    hT = jnp.transpose(h, (1, 2, 3, 0)).reshape(_NJ * _COL, _COL, B)
    tT = jnp.transpose(t, (1, 2, 0))
    vT = jnp.transpose(v, (1, 2, 0))
    kbig = jnp.asarray(_KBIG_NP)

    grid = (_NB, _NJB)
    ttT, psums, cnts = pl.pallas_call(
        _fused_kernel,
        grid=grid,
        in_specs=[
            pl.BlockSpec((_NJ, 2, _BB), lambda bi, jj: (0, 0, bi)),
            pl.BlockSpec((_NJ, 2, _BB), lambda bi, jj: (0, 0, bi)),
            pl.BlockSpec((_JPB * _COL, _COL, _BB), lambda bi, jj: (jj, 0, bi)),
            pl.BlockSpec((32, 28), lambda bi, jj: (0, 0)),
        ],
        out_specs=[
            pl.BlockSpec((_JPB * _COL, _COL, _BB), lambda bi, jj: (jj, 0, bi)),
            pl.BlockSpec((1, 1, _BB), lambda bi, jj: (bi, 0, 0)),
            pl.BlockSpec((1, 1, _BB), lambda bi, jj: (bi, 0, 0)),
        ],
        out_shape=[
            jax.ShapeDtypeStruct((_NJ * _COL, _COL, B), jnp.float32),
            jax.ShapeDtypeStruct((_NB, 1, _BB), jnp.float32),
            jax.ShapeDtypeStruct((_NB, 1, _BB), jnp.float32),
        ],
        compiler_params=pltpu.CompilerParams(
            dimension_semantics=("parallel", "arbitrary"),
        ),
    )(tT, vT, hT, kbig)

    tt = ttT.reshape(_NJ, _COL, _COL, B).transpose(3, 0, 1, 2)
    d1 = jnp.sum(psums) / jnp.sum(cnts)
    return d1, tt
